# Initial kernel scaffold; baseline (speedup 1.0000x reference)
import jax
import jax.numpy as jnp
from jax.experimental import pallas as pl


def kernel(x, edge_index, W1, b1, W2, b2):
    raise NotImplementedError("write your pallas kernel here")


# speedup vs baseline: 9.3334x; 9.3334x over previous
import functools

import jax
import jax.numpy as jnp
from jax import lax
from jax.experimental import pallas as pl
from jax.experimental.pallas import tpu as pltpu
from jax.experimental.pallas import tpu_sc as plsc

N = 10000
E = 160000
IN_C = 256
HID = 256
NCLS = 8

NP = 10240
DUMMY = N
ROWS_PT = NP // 16
EB = 128
EP = 163840
NBLK = EP // EB
QW = 64
CW = 16

_mesh = plsc.VectorSubcoreMesh(core_axis_name="c", subcore_axis_name="s")


@functools.partial(
    pl.kernel,
    out_type=jax.ShapeDtypeStruct((2, NP, CW), jnp.float32),
    mesh=_mesh,
    scratch_types=[
        pltpu.VMEM((NBLK // 32, EB), jnp.int32),
        pltpu.VMEM((EB, CW), jnp.float32),
        pltpu.VMEM((EB, CW), jnp.float32),
        pltpu.VMEM_SHARED((NP, CW), jnp.float32),
    ],
)
def _deg_kernel(dst_hbm, out_hbm, dst_buf, ones_buf, zero_buf, acc):
    c = lax.axis_index("c")
    s = lax.axis_index("s")
    blocks_per_tile = NBLK // 32

    onehot = jnp.where(lax.iota(jnp.int32, 16) == 0, 1.0, 0.0).astype(jnp.float32)
    zeros16 = jnp.zeros((16,), jnp.float32)
    for r in range(EB):
        ones_buf[r] = onehot
        zero_buf[r] = zeros16

    for k in range(ROWS_PT // EB):
        pltpu.sync_copy(zero_buf, acc.at[pl.ds(s * ROWS_PT + k * EB, EB)])

    base_blk = c * (NBLK // 2) + s * blocks_per_tile
    pltpu.sync_copy(dst_hbm.at[pl.ds(base_blk, blocks_per_tile)], dst_buf)
    plsc.subcore_barrier()

    def body(j, carry):
        pltpu.sync_copy(ones_buf, acc.at[dst_buf.at[j]], add=True)
        return carry

    lax.fori_loop(0, blocks_per_tile, body, 0)
    plsc.subcore_barrier()
    pltpu.sync_copy(acc.at[pl.ds(s * ROWS_PT, ROWS_PT)],
                    out_hbm.at[c, pl.ds(s * ROWS_PT, ROWS_PT)])


@functools.partial(
    pl.kernel,
    out_type=jax.ShapeDtypeStruct((4, NP, QW), jnp.float32),
    mesh=_mesh,
    scratch_types=[
        pltpu.VMEM((NBLK // 16, EB), jnp.int32),
        pltpu.VMEM((NBLK // 16, EB), jnp.int32),
        pltpu.VMEM((EB, QW), jnp.float32),
        pltpu.VMEM((EB, QW), jnp.float32),
        pltpu.SemaphoreType.DMA,
        pltpu.SemaphoreType.DMA,
        pltpu.VMEM_SHARED((NP, QW), jnp.float32),
    ],
    compiler_params=pltpu.CompilerParams(use_tc_tiling_on_sc=False),
)
def _agg1_kernel(hs4_hbm, src_hbm, dst_hbm, out_hbm,
                 src_buf, dst_buf, rows_a, rows_b, sem_a, sem_b, acc):
    c = lax.axis_index("c")
    s = lax.axis_index("s")
    bpt = NBLK // 16

    pltpu.sync_copy(src_hbm.at[pl.ds(s * bpt, bpt)], src_buf)
    pltpu.sync_copy(dst_hbm.at[pl.ds(s * bpt, bpt)], dst_buf)

    rows = (rows_a, rows_b)
    sems = (sem_a, sem_b)

    for p in range(2):
        q = 2 * p + c
        table = hs4_hbm.at[q]
        pltpu.sync_copy(table.at[pl.ds(s * ROWS_PT, ROWS_PT)],
                        acc.at[pl.ds(s * ROWS_PT, ROWS_PT)])
        plsc.subcore_barrier()

        pltpu.async_copy(table.at[src_buf.at[0]], rows_a, sem_a)

        def body(j, carry):
            cur = jax.lax.rem(j, 2)
            for k in range(2):
                @pl.when(cur == k)
                def _():
                    pltpu.make_async_copy(table.at[src_buf.at[j]], rows[k],
                                          sems[k]).wait()
                    @pl.when(j + 1 < bpt)
                    def _():
                        pltpu.async_copy(table.at[src_buf.at[j + 1]],
                                         rows[1 - k], sems[1 - k])
                    pltpu.sync_copy(rows[k], acc.at[dst_buf.at[j]], add=True)
            return carry

        lax.fori_loop(0, bpt, body, 0)
        plsc.subcore_barrier()
        pltpu.sync_copy(acc.at[pl.ds(s * ROWS_PT, ROWS_PT)],
                        out_hbm.at[q, pl.ds(s * ROWS_PT, ROWS_PT)])


@functools.partial(
    pl.kernel,
    out_type=jax.ShapeDtypeStruct((2, NP, CW), jnp.float32),
    mesh=_mesh,
    scratch_types=[
        pltpu.VMEM((NBLK // 32, EB), jnp.int32),
        pltpu.VMEM((NBLK // 32, EB), jnp.int32),
        pltpu.VMEM((EB, CW), jnp.float32),
        pltpu.VMEM((EB, CW), jnp.float32),
        pltpu.VMEM((EB, CW), jnp.float32),
        pltpu.SemaphoreType.DMA,
        pltpu.SemaphoreType.DMA,
        pltpu.VMEM_SHARED((NP, CW), jnp.float32),
    ],
    compiler_params=pltpu.CompilerParams(use_tc_tiling_on_sc=False),
)
def _agg2_kernel(h2s_hbm, src_hbm, dst_hbm, out_hbm,
                 src_buf, dst_buf, rows_a, rows_b, zero_buf, sem_a, sem_b, acc):
    c = lax.axis_index("c")
    s = lax.axis_index("s")
    bpt = NBLK // 32

    zeros16 = jnp.zeros((16,), jnp.float32)
    for r in range(EB):
        zero_buf[r] = zeros16

    @pl.when(c == 0)
    def _():
        pltpu.sync_copy(h2s_hbm.at[pl.ds(s * ROWS_PT, ROWS_PT)],
                        acc.at[pl.ds(s * ROWS_PT, ROWS_PT)])

    @pl.when(c == 1)
    def _():
        for k in range(ROWS_PT // EB):
            pltpu.sync_copy(zero_buf, acc.at[pl.ds(s * ROWS_PT + k * EB, EB)])

    base_blk = c * (NBLK // 2) + s * bpt
    pltpu.sync_copy(src_hbm.at[pl.ds(base_blk, bpt)], src_buf)
    pltpu.sync_copy(dst_hbm.at[pl.ds(base_blk, bpt)], dst_buf)
    plsc.subcore_barrier()

    rows = (rows_a, rows_b)
    sems = (sem_a, sem_b)
    pltpu.async_copy(h2s_hbm.at[src_buf.at[0]], rows_a, sem_a)

    def body(j, carry):
        cur = jax.lax.rem(j, 2)
        for k in range(2):
            @pl.when(cur == k)
            def _():
                pltpu.make_async_copy(h2s_hbm.at[src_buf.at[j]], rows[k],
                                      sems[k]).wait()
                @pl.when(j + 1 < bpt)
                def _():
                    pltpu.async_copy(h2s_hbm.at[src_buf.at[j + 1]],
                                     rows[1 - k], sems[1 - k])
                pltpu.sync_copy(rows[k], acc.at[dst_buf.at[j]], add=True)
        return carry

    lax.fori_loop(0, bpt, body, 0)
    plsc.subcore_barrier()
    pltpu.sync_copy(acc.at[pl.ds(s * ROWS_PT, ROWS_PT)],
                    out_hbm.at[c, pl.ds(s * ROWS_PT, ROWS_PT)])


def _dinv_from(deg2_blk):
    deg = deg2_blk[0, :, 0] + deg2_blk[1, :, 0] + 1.0
    return lax.rsqrt(deg)


def _mm1_body(x_blk, w1_blk, deg2_blk, out_blk):
    dinv = _dinv_from(deg2_blk)
    h = jnp.dot(x_blk[...], w1_blk[...], preferred_element_type=jnp.float32)
    hs = h * dinv[:, None]
    out_blk[...] = jnp.stack(
        [hs[:, q * QW:(q + 1) * QW] for q in range(4)], axis=0)


def _mm2_body(agg_blk, deg2_blk, w2_blk, b1_blk, out_blk):
    dinv = _dinv_from(deg2_blk)
    aggf = jnp.concatenate([agg_blk[q] for q in range(4)], axis=1)
    out1 = jnp.maximum(aggf * dinv[:, None] + b1_blk[0], 0.0)
    h2 = jnp.dot(out1, w2_blk[...], preferred_element_type=jnp.float32)
    out_blk[...] = h2 * dinv[:, None]


def _fin_body(agg2_blk, deg2_blk, b2_blk, out_blk):
    dinv = _dinv_from(deg2_blk)
    out_blk[...] = (agg2_blk[0] + agg2_blk[1]) * dinv[:, None] + b2_blk[0]


_RB = 512
_GRID = NP // _RB

_deg2_spec = pl.BlockSpec((2, _RB, CW), lambda i: (0, i, 0))

_mm1_call = pl.pallas_call(
    _mm1_body,
    grid=(_GRID,),
    in_specs=[
        pl.BlockSpec((_RB, IN_C), lambda i: (i, 0)),
        pl.BlockSpec((IN_C, HID), lambda i: (0, 0)),
        _deg2_spec,
    ],
    out_specs=pl.BlockSpec((4, _RB, QW), lambda i: (0, i, 0)),
    out_shape=jax.ShapeDtypeStruct((4, NP, QW), jnp.float32),
)

_mm2_call = pl.pallas_call(
    _mm2_body,
    grid=(_GRID,),
    in_specs=[
        pl.BlockSpec((4, _RB, QW), lambda i: (0, i, 0)),
        _deg2_spec,
        pl.BlockSpec((HID, CW), lambda i: (0, 0)),
        pl.BlockSpec((1, HID), lambda i: (0, 0)),
    ],
    out_specs=pl.BlockSpec((_RB, CW), lambda i: (i, 0)),
    out_shape=jax.ShapeDtypeStruct((NP, CW), jnp.float32),
)

_fin_call = pl.pallas_call(
    _fin_body,
    grid=(_GRID,),
    in_specs=[
        pl.BlockSpec((2, _RB, CW), lambda i: (0, i, 0)),
        _deg2_spec,
        pl.BlockSpec((1, CW), lambda i: (0, 0)),
    ],
    out_specs=pl.BlockSpec((_RB, CW), lambda i: (i, 0)),
    out_shape=jax.ShapeDtypeStruct((NP, CW), jnp.float32),
)


def kernel(x, edge_index, W1, b1, W2, b2):
    x_p = jnp.zeros((NP, IN_C), x.dtype).at[:N].set(x)
    pad = jnp.full((2, EP - E), DUMMY, edge_index.dtype)
    ei = jnp.concatenate([edge_index, pad], axis=1)
    src_r = ei[0].reshape(NBLK, EB)
    dst_r = ei[1].reshape(NBLK, EB)
    w2_p = jnp.zeros((HID, CW), W2.dtype).at[:, :NCLS].set(W2)
    b1_r = b1.reshape(1, HID)
    b2_p = jnp.zeros((1, CW), b2.dtype).at[0, :NCLS].set(b2)

    deg2 = _deg_kernel(dst_r)
    hs4 = _mm1_call(x_p, W1, deg2)
    agg = _agg1_kernel(hs4, src_r, dst_r)
    h2s = _mm2_call(agg, deg2, w2_p, b1_r)
    agg2 = _agg2_kernel(h2s, src_r, dst_r)
    out = _fin_call(agg2, deg2, b2_p)
    return out[:N, :NCLS]
